# Initial kernel scaffold; baseline (speedup 1.0000x reference)
import jax
import jax.numpy as jnp
from jax.experimental import pallas as pl


def kernel(x, edge_index, edge_attr, batch, prob_dist, ldxb_flag, c1_emW, c1_emb, c1_Wih, c1_Whh, c1_bih, c1_bhh, c2_emW, c2_emb, c2_Wih, c2_Whh, c2_bih, c2_bhh, gateW, gateb, expW, expb, projW, projb, fcW, fcb):
    raise NotImplementedError("write your pallas kernel here")


# speedup vs baseline: 3.0608x; 3.0608x over previous
import functools

import jax
import jax.numpy as jnp
from jax import lax
from jax.experimental import pallas as pl
from jax.experimental.pallas import tpu as pltpu
from jax.experimental.pallas import tpu_sc as plsc

N = 10000
E = 320000
D = 128
G = 16
NUM_EXPERTS = 4

NC = 2
NS = 16
NW = NC * NS
K = 128
EPW = 10240
EPAD = NW * EPW
ITERS = EPW // K
NROWS = 10240
RPT = NROWS // NS
DUMMY_ROW = N + 64


_MESH = plsc.VectorSubcoreMesh(
    core_axis_name="c", subcore_axis_name="s", num_cores=NC, num_subcores=NS)


def _sc_segment_pass():
  def body(x_hbm, src_hbm, dst_hbm, zs_hbm, s_out,
           src_v, dst_v, rows_v, s_sh, sem):
    c = lax.axis_index("c")
    s = lax.axis_index("s")
    w = c * NS + s
    row0 = s * RPT
    pltpu.sync_copy(zs_hbm.at[pl.ds(row0, RPT)], s_sh.at[pl.ds(row0, RPT)])
    plsc.subcore_barrier()

    base0 = w * EPW

    def step(i, carry):
      b = base0 + i * K
      pltpu.sync_copy(src_hbm.at[pl.ds(b, K)], src_v)
      pltpu.sync_copy(dst_hbm.at[pl.ds(b, K)], dst_v)
      pltpu.async_copy(x_hbm.at[src_v], rows_v, sem).wait()
      pltpu.sync_copy(rows_v, s_sh.at[dst_v], add=True)
      return carry

    lax.fori_loop(0, ITERS, step, 0)
    plsc.subcore_barrier()
    pltpu.sync_copy(s_sh.at[pl.ds(row0, RPT)], s_out.at[c, pl.ds(row0, RPT)])

  return pl.kernel(
      body,
      out_type=jax.ShapeDtypeStruct((NC, NROWS, D), jnp.float32),
      mesh=_MESH,
      scratch_types=[
          pltpu.VMEM((K,), jnp.int32),
          pltpu.VMEM((K,), jnp.int32),
          pltpu.VMEM((K, D), jnp.float32),
          pltpu.VMEM_SHARED((NROWS, D), jnp.float32),
          pltpu.SemaphoreType.DMA,
      ])


FR = NROWS * 8 // 128
NCH = FR // 128
FR_PT = FR // NS


def _sc_ea_pass():
  def body(eaf_hbm, dst_hbm, zf_hbm, ident_hbm, a_out,
           dst_v, eaf_v, a_acc, ident_v, a_sh):
    c = lax.axis_index("c")
    s = lax.axis_index("s")
    w = c * NS + s
    row0 = s * FR_PT
    pltpu.sync_copy(zf_hbm.at[pl.ds(row0, FR_PT)],
                    a_sh.at[pl.ds(row0, FR_PT)])
    pltpu.sync_copy(zf_hbm, a_acc)
    pltpu.sync_copy(ident_hbm, ident_v)
    plsc.subcore_barrier()
    iota = lax.iota(jnp.int32, 16)
    cols = jnp.bitwise_and(iota, 7)
    mlow = iota < 8
    mhigh = jnp.logical_not(mlow)

    def step(i, carry):
      b = w * EPW + i * K
      pltpu.sync_copy(dst_hbm.at[pl.ds(b, K)], dst_v)
      pltpu.sync_copy(eaf_hbm.at[pl.ds(b * 8, K * 8)], eaf_v)
      for q in range(K * 8 // 16):
        eidx = lax.shift_right_logical(q * 16 + iota, 3)
        rows = plsc.load_gather(dst_v, [eidx])
        flat = rows * 8 + cols
        frow = lax.shift_right_logical(flat, 7)
        fcol = jnp.bitwise_and(flat, 127)
        vals = eaf_v[pl.ds(q * 16, 16)]
        plsc.addupdate_scatter(a_acc, [frow, fcol], vals, mask=mlow)
        plsc.addupdate_scatter(a_acc, [frow, fcol], vals, mask=mhigh)
      return carry

    lax.fori_loop(0, ITERS, step, 0)
    plsc.subcore_barrier()
    for j in range(NCH):
      pltpu.sync_copy(a_acc.at[pl.ds(j * 128, 128)],
                      a_sh.at[ident_v.at[j]], add=True)
    plsc.subcore_barrier()
    pltpu.sync_copy(a_sh.at[pl.ds(row0, FR_PT)],
                    a_out.at[c, pl.ds(row0, FR_PT)])

  return pl.kernel(
      body,
      out_type=jax.ShapeDtypeStruct((NC, FR, 128), jnp.float32),
      mesh=_MESH,
      scratch_types=[
          pltpu.VMEM((K,), jnp.int32),
          pltpu.VMEM((K * 8,), jnp.float32),
          pltpu.VMEM((FR, 128), jnp.float32),
          pltpu.VMEM((NCH, 128), jnp.int32),
          pltpu.VMEM_SHARED((FR, 128), jnp.float32),
      ],
      compiler_params=pltpu.CompilerParams(needs_layout_passes=False))


_TCB = 1000
_TCG = N // _TCB


def _gru_body(sg, xr, ar, emwt, embr, wiht, whht, bihr, bhhr, out):
  x = xr[...]
  a = ar[0] + ar[1]
  cmat = jnp.dot(a[:, :4], emwt[...], preferred_element_type=jnp.float32)
  m = sg[0] + sg[1] + x + cmat + (a[:, 4:5] + 1.0) * embr[...]
  gi = jnp.dot(m, wiht[...], preferred_element_type=jnp.float32) + bihr[...]
  gh = jnp.dot(x, whht[...], preferred_element_type=jnp.float32) + bhhr[...]
  r = jax.nn.sigmoid(gi[:, :D] + gh[:, :D])
  z = jax.nn.sigmoid(gi[:, D:2 * D] + gh[:, D:2 * D])
  nn_ = jnp.tanh(gi[:, 2 * D:] + r * gh[:, 2 * D:])
  out[...] = jnp.maximum((1.0 - z) * nn_ + z * x, 0.0)


def _gru_layer(sg, x, a_aug, emwt, emb_r, wiht, whht, bih_r, bhh_r):
  return pl.pallas_call(
      _gru_body,
      grid=(_TCG,),
      in_specs=[
          pl.BlockSpec((NC, _TCB, D), lambda i: (0, i, 0)),
          pl.BlockSpec((_TCB, D), lambda i: (i, 0)),
          pl.BlockSpec((NC, _TCB, 8), lambda i: (0, i, 0)),
          pl.BlockSpec((4, D), lambda i: (0, 0)),
          pl.BlockSpec((1, D), lambda i: (0, 0)),
          pl.BlockSpec((D, 3 * D), lambda i: (0, 0)),
          pl.BlockSpec((D, 3 * D), lambda i: (0, 0)),
          pl.BlockSpec((1, 3 * D), lambda i: (0, 0)),
          pl.BlockSpec((1, 3 * D), lambda i: (0, 0)),
      ],
      out_specs=pl.BlockSpec((_TCB, D), lambda i: (i, 0)),
      out_shape=jax.ShapeDtypeStruct((N, D), jnp.float32),
  )(sg, x, a_aug, emwt, emb_r, wiht, whht, bih_r, bhh_r)


def _moe_pool_body(h2r, batchr, gwt, gbr, ewt, ebr, outr):
  h2 = h2r[...]
  g = jnp.dot(h2, gwt[...], preferred_element_type=jnp.float32) + gbr[...]
  g = g - jnp.max(g, axis=1, keepdims=True)
  eg = jnp.exp(g)
  gw = eg / jnp.sum(eg, axis=1, keepdims=True)
  acc = jnp.zeros((_TCB, D), jnp.float32)
  for e in range(NUM_EXPERTS):
    ye = jnp.dot(h2, ewt[e], preferred_element_type=jnp.float32) + ebr[e][None, :]
    acc = acc + gw[:, e:e + 1] * jnp.maximum(ye, 0.0)
  bt = batchr[0, 0, :]
  gidx = lax.broadcasted_iota(jnp.int32, (G, _TCB), 0)
  onehot_t = (gidx == bt[None, :]).astype(jnp.float32)
  s_part = jnp.dot(onehot_t, acc, preferred_element_type=jnp.float32)
  c_part = jnp.dot(onehot_t, jnp.ones((_TCB, D), jnp.float32),
                   preferred_element_type=jnp.float32)

  @pl.when(pl.program_id(0) == 0)
  def _():
    outr[...] = jnp.zeros((2 * G, D), jnp.float32)

  outr[0:G, :] += s_part
  outr[G:2 * G, :] += c_part


def _moe_pool(h2, batch3, gwt, gb_r, ewt, eb):
  return pl.pallas_call(
      _moe_pool_body,
      grid=(_TCG,),
      in_specs=[
          pl.BlockSpec((_TCB, D), lambda i: (i, 0)),
          pl.BlockSpec((1, 1, _TCB), lambda i: (i, 0, 0)),
          pl.BlockSpec((D, NUM_EXPERTS), lambda i: (0, 0)),
          pl.BlockSpec((1, NUM_EXPERTS), lambda i: (0, 0)),
          pl.BlockSpec((NUM_EXPERTS, D, D), lambda i: (0, 0, 0)),
          pl.BlockSpec((NUM_EXPERTS, D), lambda i: (0, 0)),
      ],
      out_specs=pl.BlockSpec((2 * G, D), lambda i: (0, 0)),
      out_shape=jax.ShapeDtypeStruct((2 * G, D), jnp.float32),
  )(h2, batch3, gwt, gb_r, ewt, eb)


def _final_body(pr, flr, pwr, pbr, w1r, w2r, fbr, ger, lor):
  s = pr[0:G, :]
  cnt = pr[G:2 * G, :]
  ge = s / jnp.maximum(cnt, 1.0)
  ldxb = flr[:, 0:1] * pwr[...] + pbr[...]
  logits = (jnp.dot(ldxb, w1r[...], preferred_element_type=jnp.float32)
            + jnp.dot(ge, w2r[...], preferred_element_type=jnp.float32)
            + fbr[...])
  ger[...] = ge
  lor[...] = logits


def _final(pooled, flag_b, pjw_r, pjb_r, w1t, w2t, fcb_r):
  return pl.pallas_call(
      _final_body,
      out_shape=(jax.ShapeDtypeStruct((G, D), jnp.float32),
                 jax.ShapeDtypeStruct((G, D), jnp.float32)),
  )(pooled, flag_b, pjw_r, pjb_r, w1t, w2t, fcb_r)


def kernel(x, edge_index, edge_attr, batch, prob_dist, ldxb_flag, c1_emW,
           c1_emb, c1_Wih, c1_Whh, c1_bih, c1_bhh, c2_emW, c2_emb, c2_Wih,
           c2_Whh, c2_bih, c2_bhh, gateW, gateb, expW, expb, projW, projb,
           fcW, fcb):
  f32 = jnp.float32
  pad = EPAD - E
  src = jnp.concatenate(
      [edge_index[0].astype(jnp.int32), jnp.zeros((pad,), jnp.int32)])
  dst = jnp.concatenate(
      [edge_index[1].astype(jnp.int32),
       jnp.full((pad,), DUMMY_ROW, jnp.int32)])
  ea_aug = jnp.concatenate(
      [edge_attr.astype(f32),
       jnp.ones((E, 1), f32),
       jnp.zeros((E, 3), f32)], axis=1)
  ea_aug = jnp.concatenate([ea_aug, jnp.zeros((pad, 8), f32)], axis=0)
  zs = jnp.zeros((NROWS, D), f32)
  zf = jnp.zeros((FR, D), f32)
  ident = jnp.arange(FR, dtype=jnp.int32).reshape(NCH, 128)

  a_flat = _sc_ea_pass()(ea_aug.reshape(-1), dst, zf, ident)
  a_aug = a_flat.reshape(NC, NROWS, 8)
  s1 = _sc_segment_pass()(x.astype(f32), src, dst, zs)

  emwt1 = c1_emW.T.astype(f32)
  emwt2 = c2_emW.T.astype(f32)
  h1 = _gru_layer(s1, x.astype(f32), a_aug, emwt1, c1_emb[None, :],
                  c1_Wih.T, c1_Whh.T, c1_bih[None, :], c1_bhh[None, :])

  s2 = _sc_segment_pass()(h1, src, dst, zs)
  h2 = _gru_layer(s2, h1, a_aug, emwt2, c2_emb[None, :],
                  c2_Wih.T, c2_Whh.T, c2_bih[None, :], c2_bhh[None, :])

  batch3 = batch.astype(jnp.int32).reshape(_TCG, 1, _TCB)
  pooled = _moe_pool(h2, batch3, gateW.T, gateb[None, :],
                     jnp.transpose(expW, (0, 2, 1)), expb)

  flag_b = jnp.broadcast_to(ldxb_flag.astype(f32)[:, None], (G, D))
  pjw_r = jnp.zeros((1, D), f32).at[0, :100].set(projW[:, 0])
  pjb_r = jnp.zeros((1, D), f32).at[0, :100].set(projb)
  w1t = jnp.zeros((D, D), f32).at[:100, :].set(fcW[:, :100].T)
  w2t = fcW[:, 100:].T
  ge, logits = _final(pooled, flag_b, pjw_r, pjb_r, w1t, w2t, fcb[None, :])
  return (ge, ge, logits)


# speedup vs baseline: 3.3403x; 1.0913x over previous
import functools

import jax
import jax.numpy as jnp
from jax import lax
from jax.experimental import pallas as pl
from jax.experimental.pallas import tpu as pltpu
from jax.experimental.pallas import tpu_sc as plsc

N = 10000
E = 320000
D = 128
G = 16
NUM_EXPERTS = 4

NC = 2
NS = 16
NW = NC * NS
K = 128
EPW = 10240
EPAD = NW * EPW
ITERS = EPW // K
NROWS = 10240
RPT = NROWS // NS
DUMMY_ROW = N + 64
HALVES = 2
CPH = EPW // K // HALVES


_MESH = plsc.VectorSubcoreMesh(
    core_axis_name="c", subcore_axis_name="s", num_cores=NC, num_subcores=NS)


def _sc_segment_pass():
  def body(x_hbm, src_hbm, dst_hbm, zs_hbm, s_out,
           srcb_v, dstb_v, rows0, rows1, s_sh, gs0, gs1, ss0, ss1):
    c = lax.axis_index("c")
    s = lax.axis_index("s")
    w = c * NS + s
    row0 = s * RPT
    pltpu.sync_copy(zs_hbm.at[pl.ds(row0, RPT)], s_sh.at[pl.ds(row0, RPT)])
    plsc.subcore_barrier()

    for h in range(HALVES):
      pltpu.sync_copy(src_hbm.at[w, h], srcb_v)
      pltpu.sync_copy(dst_hbm.at[w, h], dstb_v)

      def pair(j, carry):
        g0 = pltpu.async_copy(x_hbm.at[srcb_v.at[2 * j]], rows0, gs0)
        g1 = pltpu.async_copy(x_hbm.at[srcb_v.at[2 * j + 1]], rows1, gs1)
        g0.wait()
        s0 = pltpu.async_copy(rows0, s_sh.at[dstb_v.at[2 * j]], ss0, add=True)
        g1.wait()
        s1 = pltpu.async_copy(rows1, s_sh.at[dstb_v.at[2 * j + 1]], ss1,
                              add=True)
        s0.wait()
        s1.wait()
        return carry

      lax.fori_loop(0, CPH // 2, pair, 0)
    plsc.subcore_barrier()
    pltpu.sync_copy(s_sh.at[pl.ds(row0, RPT)], s_out.at[c, pl.ds(row0, RPT)])

  return pl.kernel(
      body,
      out_type=jax.ShapeDtypeStruct((NC, NROWS, D), jnp.float32),
      mesh=_MESH,
      scratch_types=[
          pltpu.VMEM((CPH, K), jnp.int32),
          pltpu.VMEM((CPH, K), jnp.int32),
          pltpu.VMEM((K, D), jnp.float32),
          pltpu.VMEM((K, D), jnp.float32),
          pltpu.VMEM_SHARED((NROWS, D), jnp.float32),
          pltpu.SemaphoreType.DMA,
          pltpu.SemaphoreType.DMA,
          pltpu.SemaphoreType.DMA,
          pltpu.SemaphoreType.DMA,
      ])


FR = NROWS * 8 // 128
NCH = FR // 128
FR_PT = FR // NS


def _sc_ea_pass():
  def body(eaf_hbm, dst_hbm, zf_hbm, ident_hbm, a_out,
           dst_v, eaf_v, a_acc, ident_v, a_sh):
    c = lax.axis_index("c")
    s = lax.axis_index("s")
    w = c * NS + s
    row0 = s * FR_PT
    pltpu.sync_copy(zf_hbm.at[pl.ds(row0, FR_PT)],
                    a_sh.at[pl.ds(row0, FR_PT)])
    pltpu.sync_copy(zf_hbm, a_acc)
    pltpu.sync_copy(ident_hbm, ident_v)
    plsc.subcore_barrier()
    iota = lax.iota(jnp.int32, 16)
    cols = jnp.bitwise_and(iota, 7)
    mlow = iota < 8
    mhigh = jnp.logical_not(mlow)

    def step(i, carry):
      b = w * EPW + i * K
      pltpu.sync_copy(dst_hbm.at[pl.ds(b, K)], dst_v)
      pltpu.sync_copy(eaf_hbm.at[pl.ds(b * 8, K * 8)], eaf_v)
      for q in range(K * 8 // 16):
        eidx = lax.shift_right_logical(q * 16 + iota, 3)
        rows = plsc.load_gather(dst_v, [eidx])
        flat = rows * 8 + cols
        frow = lax.shift_right_logical(flat, 7)
        fcol = jnp.bitwise_and(flat, 127)
        vals = eaf_v[pl.ds(q * 16, 16)]
        plsc.addupdate_scatter(a_acc, [frow, fcol], vals, mask=mlow)
        plsc.addupdate_scatter(a_acc, [frow, fcol], vals, mask=mhigh)
      return carry

    lax.fori_loop(0, ITERS, step, 0)
    plsc.subcore_barrier()
    for j in range(NCH):
      pltpu.sync_copy(a_acc.at[pl.ds(j * 128, 128)],
                      a_sh.at[ident_v.at[j]], add=True)
    plsc.subcore_barrier()
    pltpu.sync_copy(a_sh.at[pl.ds(row0, FR_PT)],
                    a_out.at[c, pl.ds(row0, FR_PT)])

  return pl.kernel(
      body,
      out_type=jax.ShapeDtypeStruct((NC, FR, 128), jnp.float32),
      mesh=_MESH,
      scratch_types=[
          pltpu.VMEM((K,), jnp.int32),
          pltpu.VMEM((K * 8,), jnp.float32),
          pltpu.VMEM((FR, 128), jnp.float32),
          pltpu.VMEM((NCH, 128), jnp.int32),
          pltpu.VMEM_SHARED((FR, 128), jnp.float32),
      ],
      compiler_params=pltpu.CompilerParams(needs_layout_passes=False))


_TCB = 1000
_TCG = N // _TCB


def _gru_body(sg, xr, ar, emwt, embr, wiht, whht, bihr, bhhr, out):
  x = xr[...]
  a = ar[0] + ar[1]
  cmat = jnp.dot(a[:, :4], emwt[...], preferred_element_type=jnp.float32)
  m = sg[0] + sg[1] + x + cmat + (a[:, 4:5] + 1.0) * embr[...]
  gi = jnp.dot(m, wiht[...], preferred_element_type=jnp.float32) + bihr[...]
  gh = jnp.dot(x, whht[...], preferred_element_type=jnp.float32) + bhhr[...]
  r = jax.nn.sigmoid(gi[:, :D] + gh[:, :D])
  z = jax.nn.sigmoid(gi[:, D:2 * D] + gh[:, D:2 * D])
  nn_ = jnp.tanh(gi[:, 2 * D:] + r * gh[:, 2 * D:])
  out[...] = jnp.maximum((1.0 - z) * nn_ + z * x, 0.0)


def _gru_layer(sg, x, a_aug, emwt, emb_r, wiht, whht, bih_r, bhh_r):
  return pl.pallas_call(
      _gru_body,
      grid=(_TCG,),
      in_specs=[
          pl.BlockSpec((NC, _TCB, D), lambda i: (0, i, 0)),
          pl.BlockSpec((_TCB, D), lambda i: (i, 0)),
          pl.BlockSpec((NC, _TCB, 8), lambda i: (0, i, 0)),
          pl.BlockSpec((4, D), lambda i: (0, 0)),
          pl.BlockSpec((1, D), lambda i: (0, 0)),
          pl.BlockSpec((D, 3 * D), lambda i: (0, 0)),
          pl.BlockSpec((D, 3 * D), lambda i: (0, 0)),
          pl.BlockSpec((1, 3 * D), lambda i: (0, 0)),
          pl.BlockSpec((1, 3 * D), lambda i: (0, 0)),
      ],
      out_specs=pl.BlockSpec((_TCB, D), lambda i: (i, 0)),
      out_shape=jax.ShapeDtypeStruct((N, D), jnp.float32),
  )(sg, x, a_aug, emwt, emb_r, wiht, whht, bih_r, bhh_r)


def _moe_pool_body(h2r, batchr, gwt, gbr, ewt, ebr, outr):
  h2 = h2r[...]
  g = jnp.dot(h2, gwt[...], preferred_element_type=jnp.float32) + gbr[...]
  g = g - jnp.max(g, axis=1, keepdims=True)
  eg = jnp.exp(g)
  gw = eg / jnp.sum(eg, axis=1, keepdims=True)
  acc = jnp.zeros((_TCB, D), jnp.float32)
  for e in range(NUM_EXPERTS):
    ye = jnp.dot(h2, ewt[e], preferred_element_type=jnp.float32) + ebr[e][None, :]
    acc = acc + gw[:, e:e + 1] * jnp.maximum(ye, 0.0)
  bt = batchr[0, 0, :]
  gidx = lax.broadcasted_iota(jnp.int32, (G, _TCB), 0)
  onehot_t = (gidx == bt[None, :]).astype(jnp.float32)
  s_part = jnp.dot(onehot_t, acc, preferred_element_type=jnp.float32)
  c_part = jnp.dot(onehot_t, jnp.ones((_TCB, D), jnp.float32),
                   preferred_element_type=jnp.float32)

  @pl.when(pl.program_id(0) == 0)
  def _():
    outr[...] = jnp.zeros((2 * G, D), jnp.float32)

  outr[0:G, :] += s_part
  outr[G:2 * G, :] += c_part


def _moe_pool(h2, batch3, gwt, gb_r, ewt, eb):
  return pl.pallas_call(
      _moe_pool_body,
      grid=(_TCG,),
      in_specs=[
          pl.BlockSpec((_TCB, D), lambda i: (i, 0)),
          pl.BlockSpec((1, 1, _TCB), lambda i: (i, 0, 0)),
          pl.BlockSpec((D, NUM_EXPERTS), lambda i: (0, 0)),
          pl.BlockSpec((1, NUM_EXPERTS), lambda i: (0, 0)),
          pl.BlockSpec((NUM_EXPERTS, D, D), lambda i: (0, 0, 0)),
          pl.BlockSpec((NUM_EXPERTS, D), lambda i: (0, 0)),
      ],
      out_specs=pl.BlockSpec((2 * G, D), lambda i: (0, 0)),
      out_shape=jax.ShapeDtypeStruct((2 * G, D), jnp.float32),
  )(h2, batch3, gwt, gb_r, ewt, eb)


def _final_body(pr, flr, pwr, pbr, w1r, w2r, fbr, ger, lor):
  s = pr[0:G, :]
  cnt = pr[G:2 * G, :]
  ge = s / jnp.maximum(cnt, 1.0)
  ldxb = flr[:, 0:1] * pwr[...] + pbr[...]
  logits = (jnp.dot(ldxb, w1r[...], preferred_element_type=jnp.float32)
            + jnp.dot(ge, w2r[...], preferred_element_type=jnp.float32)
            + fbr[...])
  ger[...] = ge
  lor[...] = logits


def _final(pooled, flag_b, pjw_r, pjb_r, w1t, w2t, fcb_r):
  return pl.pallas_call(
      _final_body,
      out_shape=(jax.ShapeDtypeStruct((G, D), jnp.float32),
                 jax.ShapeDtypeStruct((G, D), jnp.float32)),
  )(pooled, flag_b, pjw_r, pjb_r, w1t, w2t, fcb_r)


def kernel(x, edge_index, edge_attr, batch, prob_dist, ldxb_flag, c1_emW,
           c1_emb, c1_Wih, c1_Whh, c1_bih, c1_bhh, c2_emW, c2_emb, c2_Wih,
           c2_Whh, c2_bih, c2_bhh, gateW, gateb, expW, expb, projW, projb,
           fcW, fcb):
  f32 = jnp.float32
  pad = EPAD - E
  src = jnp.concatenate(
      [edge_index[0].astype(jnp.int32), jnp.zeros((pad,), jnp.int32)])
  dst = jnp.concatenate(
      [edge_index[1].astype(jnp.int32),
       jnp.full((pad,), DUMMY_ROW, jnp.int32)])
  ea_aug = jnp.concatenate(
      [edge_attr.astype(f32),
       jnp.ones((E, 1), f32),
       jnp.zeros((E, 3), f32)], axis=1)
  ea_aug = jnp.concatenate([ea_aug, jnp.zeros((pad, 8), f32)], axis=0)
  zs = jnp.zeros((NROWS, D), f32)
  zf = jnp.zeros((FR, D), f32)
  ident = jnp.arange(FR, dtype=jnp.int32).reshape(NCH, 128)

  src4 = src.reshape(NW, HALVES, CPH, K)
  dst4 = dst.reshape(NW, HALVES, CPH, K)

  a_flat = _sc_ea_pass()(ea_aug.reshape(-1), dst, zf, ident)
  a_aug = a_flat.reshape(NC, NROWS, 8)
  s1 = _sc_segment_pass()(x.astype(f32), src4, dst4, zs)

  emwt1 = c1_emW.T.astype(f32)
  emwt2 = c2_emW.T.astype(f32)
  h1 = _gru_layer(s1, x.astype(f32), a_aug, emwt1, c1_emb[None, :],
                  c1_Wih.T, c1_Whh.T, c1_bih[None, :], c1_bhh[None, :])

  s2 = _sc_segment_pass()(h1, src4, dst4, zs)
  h2 = _gru_layer(s2, h1, a_aug, emwt2, c2_emb[None, :],
                  c2_Wih.T, c2_Whh.T, c2_bih[None, :], c2_bhh[None, :])

  batch3 = batch.astype(jnp.int32).reshape(_TCG, 1, _TCB)
  pooled = _moe_pool(h2, batch3, gateW.T, gateb[None, :],
                     jnp.transpose(expW, (0, 2, 1)), expb)

  flag_b = jnp.broadcast_to(ldxb_flag.astype(f32)[:, None], (G, D))
  pjw_r = jnp.zeros((1, D), f32).at[0, :100].set(projW[:, 0])
  pjb_r = jnp.zeros((1, D), f32).at[0, :100].set(projb)
  w1t = jnp.zeros((D, D), f32).at[:100, :].set(fcW[:, :100].T)
  w2t = fcW[:, 100:].T
  ge, logits = _final(pooled, flag_b, pjw_r, pjb_r, w1t, w2t, fcb[None, :])
  return (ge, ge, logits)
